# baseline (speedup 1.0000x reference)
import functools

import jax
import jax.numpy as jnp
from jax.experimental import pallas as pl
from jax.experimental.pallas import tpu as pltpu
from jax.experimental.pallas import tpu_sc as plsc

_K = 64
_KP = 128
_D = 128
_BLK = 16384
_SC_CHUNK = 128


def _vq_idx_body(x_ref, emb2_ref, esq_ref, idx_ref, loss_ref):
    i = pl.program_id(0)
    x = x_ref[...]
    emb2 = emb2_ref[...]
    esq = esq_ref[...]
    scores2 = jax.lax.dot_general(x, emb2, (((1,), (1,)), ((), ())),
                                  preferred_element_type=jnp.float32)
    ones = jnp.ones((_KP, _D), jnp.float32)
    xsq = jax.lax.dot_general(x * x, ones, (((1,), (1,)), ((), ())),
                              preferred_element_type=jnp.float32)
    dist = (xsq + esq) - scores2
    min_val = jnp.min(dist, axis=1, keepdims=True)
    iota = jax.lax.broadcasted_iota(jnp.int32, dist.shape, 1).astype(jnp.float32)
    masked = jnp.where(dist <= min_val, iota, float(_KP))
    idx = jnp.min(masked, axis=1, keepdims=True)
    idx_ref[...] = idx.astype(jnp.int32)
    part = jnp.sum(min_val)

    @pl.when(i == 0)
    def _init():
        loss_ref[0, 0] = 0.0

    loss_ref[0, 0] += part


def _argmin_indices(inputs, emb2_pad, esq_pad):
    n, d = inputs.shape
    grid = (n // _BLK,)
    idx, loss = pl.pallas_call(
        _vq_idx_body,
        grid=grid,
        in_specs=[
            pl.BlockSpec((_BLK, d), lambda i: (i, 0)),
            pl.BlockSpec((_KP, d), lambda i: (0, 0)),
            pl.BlockSpec((1, _KP), lambda i: (0, 0)),
        ],
        out_specs=[
            pl.BlockSpec((_BLK, 1), lambda i: (i, 0)),
            pl.BlockSpec(memory_space=pltpu.SMEM),
        ],
        out_shape=[
            jax.ShapeDtypeStruct((n, 1), jnp.int32),
            jax.ShapeDtypeStruct((1, 1), jnp.float32),
        ],
    )(inputs, emb2_pad, esq_pad)
    return idx, loss


def _make_sc_gather(n, d, nc, ns):
    nw = nc * ns
    rows_pw = n // nw
    nch = rows_pw // _SC_CHUNK
    mesh = plsc.VectorSubcoreMesh(core_axis_name="c", subcore_axis_name="s")

    @functools.partial(
        pl.kernel,
        out_type=jax.ShapeDtypeStruct((n, d), jnp.float32),
        mesh=mesh,
        scratch_types=[
            pltpu.VMEM((nch, _SC_CHUNK), jnp.int32),
            pltpu.VMEM((_SC_CHUNK, d), jnp.float32),
            pltpu.SemaphoreType.DMA,
        ],
    )
    def gather_k(table_hbm, idx_hbm, out_hbm, idx_v, rows_v, sem):
        wid = jax.lax.axis_index("s") * nc + jax.lax.axis_index("c")
        pltpu.sync_copy(idx_hbm.at[pl.ds(wid * nch, nch)], idx_v)

        def body(g, carry):
            pltpu.async_copy(table_hbm.at[idx_v.at[g]], rows_v, sem).wait()
            base_row = (wid * nch + g) * _SC_CHUNK
            pltpu.sync_copy(rows_v, out_hbm.at[pl.ds(base_row, _SC_CHUNK)])
            return carry

        jax.lax.fori_loop(0, nch, body, 0)

    return gather_k


def kernel(inputs, embeddings):
    n, d = inputs.shape
    pad = jnp.zeros((_KP - _K, d), jnp.float32)
    emb2_pad = jnp.concatenate([2.0 * embeddings, pad], axis=0)
    esq = jnp.sum(embeddings ** 2, axis=1)
    esq_pad = jnp.concatenate(
        [esq, jnp.full((_KP - _K,), jnp.inf, jnp.float32)]).reshape(1, _KP)

    idx, loss = _argmin_indices(inputs, emb2_pad, esq_pad)

    info = plsc.get_sparse_core_info()
    gather_k = _make_sc_gather(n, d, info.num_cores, info.num_subcores)
    idx_rs = idx.reshape(n // _SC_CHUNK, _SC_CHUNK)
    q = gather_k(embeddings, idx_rs)

    vq_loss = (2.0 / (n * d)) * loss[0, 0]
    return (q, vq_loss)
